# baseline (speedup 1.0000x reference)
import functools
import numpy as np
import jax
import jax.numpy as jnp
from jax import lax
from jax.experimental import pallas as pl
from jax.experimental.pallas import tpu as pltpu
from jax.experimental.pallas import tpu_sc as plsc

N = 10000
E = 160000
HID = 128
HEADS = 16
DH = 8
NRG = 20
EFD = 4
RFD = NRG * 4

BE = 2000
BN = 2000

_OFF = np.linspace(0.0, 10.0, NRG).astype(np.float32)
_COEFF = np.float32(-0.5 / (_OFF[1] - _OFF[0]) ** 2)
_A4 = np.zeros((EFD, RFD), np.float32)
_G20 = np.zeros((NRG, RFD), np.float32)
for _a in range(EFD):
    for _g in range(NRG):
        _A4[_a, _a * NRG + _g] = 1.0
        _G20[_g, _a * NRG + _g] = 1.0
_HS = np.zeros((HID, HEADS), np.float32)
_EXH = np.zeros((HEADS, HID), np.float32)
for _h in range(HEADS):
    for _j in range(DH):
        _HS[_h * DH + _j, _h] = 1.0
        _EXH[_h, _h * DH + _j] = 1.0
_P48 = np.zeros((HEADS, 48), np.float32)
_Q48 = np.zeros((16, 48), np.float32)
_M48 = np.zeros((48, 8), np.float32)
for _h in range(HEADS):
    for _c in range(3):
        _P48[_h, 3 * _h + _c] = 1.0
        _Q48[_c, 3 * _h + _c] = 1.0
        _M48[3 * _h + _c, _c] = 1.0 / HEADS
_ISQ = np.float32(1.0 / np.sqrt(DH))


_CHW = 128
_NCH = E // _CHW
_NW = 32
_NB = 40
_IDXP = 1264
_EOUT = _NCH * _CHW + _CHW

_SC_MESH = dict(core_axis_name="c", subcore_axis_name="s")
_SC_PARAMS = pltpu.CompilerParams(use_tc_tiling_on_sc=False)


def _worker_start(wid):
    return 39 * wid + jnp.minimum(wid, 2)


def _sc_gather_multi(tables, sels, dstm, srcm):
    nt = len(tables)
    ds = [int(t.shape[1]) for t in tables]
    dts = [t.dtype for t in tables]

    @functools.partial(
        pl.kernel,
        out_type=tuple(jax.ShapeDtypeStruct((_EOUT, d), dt)
                       for d, dt in zip(ds, dts)),
        mesh=plsc.VectorSubcoreMesh(**_SC_MESH),
        compiler_params=_SC_PARAMS,
        scratch_types=(
            [pltpu.VMEM((_NB, _CHW), jnp.int32)] * 2
            + [pltpu.VMEM((_CHW, d), dt)
               for d, dt in zip(ds, dts) for _ in (0, 1)]
            + [pltpu.SemaphoreType.DMA] * (4 * nt)
        ),
    )
    def k(*refs):
        tabs = refs[0:nt]
        outs = refs[nt + 2:2 * nt + 2]
        idxa = refs[2 * nt + 2:2 * nt + 4]
        rows = [refs[2 * nt + 4 + 2 * t:2 * nt + 6 + 2 * t]
                for t in range(nt)]
        gs = [refs[4 * nt + 4 + 2 * t:4 * nt + 6 + 2 * t]
              for t in range(nt)]
        osm = [refs[6 * nt + 4 + 2 * t:6 * nt + 6 + 2 * t]
               for t in range(nt)]
        wid = lax.axis_index("s") * 2 + lax.axis_index("c")
        start = _worker_start(wid)
        pltpu.sync_copy(refs[nt].at[pl.ds(start, _NB)], idxa[0])
        pltpu.sync_copy(refs[nt + 1].at[pl.ds(start, _NB)], idxa[1])
        for t in range(nt):
            pltpu.async_copy(tabs[t].at[idxa[sels[t]].at[0]],
                             rows[t][0], gs[t][0])

        def pair(jj, carry):
            for bb in (0, 1):
                j = jj * 2 + bb
                ob = 1 - bb
                for t in range(nt):
                    pltpu.make_async_copy(
                        tabs[t].at[idxa[sels[t]].at[j]],
                        rows[t][bb], gs[t][bb]).wait()
                    pltpu.async_copy(
                        rows[t][bb],
                        outs[t].at[pl.ds((start + j) * _CHW, _CHW)],
                        osm[t][bb])

                @pl.when(j < _NB - 1)
                def _():
                    @pl.when(j >= 1)
                    def _():
                        for t in range(nt):
                            pltpu.make_async_copy(
                                rows[t][ob], outs[t].at[pl.ds(0, _CHW)],
                                osm[t][ob]).wait()
                    for t in range(nt):
                        pltpu.async_copy(
                            tabs[t].at[idxa[sels[t]].at[j + 1]],
                            rows[t][ob], gs[t][ob])
            return carry

        lax.fori_loop(0, _NB // 2, pair, 0)
        for bb in (0, 1):
            for t in range(nt):
                pltpu.make_async_copy(
                    rows[t][bb], outs[t].at[pl.ds(0, _CHW)],
                    osm[t][bb]).wait()

    return k(*tables, dstm, srcm)


_HN = N // 2
_TB = _HN + 8
_NB2 = 79


def _sc_scatter_split(exw, wv, idxm, d):
    @functools.partial(
        pl.kernel,
        out_type=(jax.ShapeDtypeStruct((N, 16), jnp.float32),
                  jax.ShapeDtypeStruct((N, d), jnp.float32)),
        mesh=plsc.VectorSubcoreMesh(**_SC_MESH),
        compiler_params=_SC_PARAMS,
        scratch_types=[
            pltpu.VMEM((_NB2, _CHW), jnp.int32),
            pltpu.VMEM((_CHW,), jnp.int32),
            pltpu.VMEM((_CHW, 16), jnp.float32),
            pltpu.VMEM((_CHW, 16), jnp.float32),
            pltpu.VMEM((_CHW, d), jnp.float32),
            pltpu.VMEM((_CHW, d), jnp.float32),
            pltpu.VMEM_SHARED((_TB, 16), jnp.float32),
            pltpu.VMEM_SHARED((_TB, d), jnp.float32),
            pltpu.SemaphoreType.DMA,
            pltpu.SemaphoreType.DMA,
        ],
    )
    def k(ex_hbm, wv_hbm, idxm_hbm, zex_hbm, zwv_hbm, oex_hbm, owv_hbm,
          idxa, idxc, exb0, exb1, wvb0, wvb1, tex, twv, ls0, ls1):
        cid = lax.axis_index("c")
        sid = lax.axis_index("s")
        base = cid * _HN
        c0 = sid * _NB2
        nch = jnp.minimum(_NB2, _NCH - c0)
        exb = (exb0, exb1)
        wvb = (wvb0, wvb1)
        ls = (ls0, ls1)

        def load(j, bb):
            pltpu.async_copy(
                ex_hbm.at[pl.ds((c0 + j) * _CHW, _CHW)], exb[bb], ls[bb])
            pltpu.async_copy(
                wv_hbm.at[pl.ds((c0 + j) * _CHW, _CHW)], wvb[bb], ls[bb])

        def load_wait(bb):
            pltpu.make_async_copy(
                ex_hbm.at[pl.ds(0, _CHW)], exb[bb], ls[bb]).wait()
            pltpu.make_async_copy(
                wv_hbm.at[pl.ds(0, _CHW)], wvb[bb], ls[bb]).wait()

        pltpu.sync_copy(idxm_hbm.at[pl.ds(c0, _NB2)], idxa)
        load(0, 0)

        @pl.when(sid == 0)
        def _init():
            pltpu.sync_copy(zex_hbm, tex)
            pltpu.sync_copy(zwv_hbm, twv)

        plsc.subcore_barrier()

        def pair(jj, carry):
            for bb in (0, 1):
                j = jj * 2 + bb

                @pl.when(j < nch)
                def _():
                    load_wait(bb)

                    @pl.when(j + 1 < nch)
                    def _():
                        load(j + 1, 1 - bb)

                    for l in range(_CHW // 16):
                        iv = idxa[j, pl.ds(l * 16, 16)]
                        inr = (iv >= base) & (iv < base + _HN)
                        idxc[pl.ds(l * 16, 16)] = jnp.where(
                            inr, iv - base, _HN)
                    pltpu.sync_copy(exb[bb], tex.at[idxc], add=True)
                    pltpu.sync_copy(wvb[bb], twv.at[idxc], add=True)
            return carry

        lax.fori_loop(0, (_NB2 + 1) // 2, pair, 0)
        plsc.subcore_barrier()

        @pl.when(sid == 0)
        def _dump():
            pltpu.sync_copy(tex.at[pl.ds(0, _HN)],
                            oex_hbm.at[pl.ds(base, _HN)])
            pltpu.sync_copy(twv.at[pl.ds(0, _HN)],
                            owv_hbm.at[pl.ds(base, _HN)])

    zex = jnp.zeros((_TB, 16), jnp.float32)
    zwv = jnp.zeros((_TB, d), jnp.float32)
    return k(exw, wv, idxm, zex, zwv)


def _ln(v, g, b):
    mu = jnp.mean(v, -1, keepdims=True)
    var = jnp.mean((v - mu) ** 2, -1, keepdims=True)
    return (v - mu) / jnp.sqrt(var + 1e-5) * g + b


def _dot(a, b):
    return jnp.dot(a, b, preferred_element_type=jnp.float32)


def _mlp128_body(h_ref, w1, b1, g1, be1, w2, b2, o_ref):
    v = _dot(h_ref[...], w1[...]) + b1[...]
    v = jnp.maximum(_ln(v, g1[...], be1[...]), 0.0)
    o_ref[...] = _dot(v, w2[...]) + b2[...]


def _mlp128(hin, p):
    return pl.pallas_call(
        _mlp128_body,
        grid=(N // BN,),
        in_specs=[
            pl.BlockSpec((BN, HID), lambda i: (i, 0)),
            pl.BlockSpec((HID, HID), lambda i: (0, 0)),
            pl.BlockSpec((1, HID), lambda i: (0, 0)),
            pl.BlockSpec((1, HID), lambda i: (0, 0)),
            pl.BlockSpec((1, HID), lambda i: (0, 0)),
            pl.BlockSpec((HID, HID), lambda i: (0, 0)),
            pl.BlockSpec((1, HID), lambda i: (0, 0)),
        ],
        out_specs=pl.BlockSpec((BN, HID), lambda i: (i, 0)),
        out_shape=jax.ShapeDtypeStruct((N, HID), jnp.float32),
    )(hin, p['W1'], p['b1'][None, :], p['g'][None, :], p['be'][None, :],
      p['W2'], p['b2'][None, :])


def _rfeat(ea, rel, off, a4, g20):
    d2 = jnp.sum(rel * rel, -1, keepdims=True)
    dist = jnp.sqrt(d2)
    sm = jnp.exp(_COEFF * (dist - off) ** 2)
    return _dot(ea, a4) * _dot(sm, g20)


def _edge1_body(ea_ref, dd_ref, ss_ref, xd_ref, xs_ref,
                off, a4, g20, hs_m, exh,
                w1e, w1r, w1d, w1s, b1,
                gk, bek, w2k, b2k, gv, bev, w2v, b2v,
                eww, ewb,
                ex_ref, wv_ref, rel_ref):
    ea = ea_ref[...]
    hd = dd_ref[:, 0:HID]
    qd = dd_ref[:, HID:2 * HID].astype(jnp.float32)
    hs = ss_ref[...]
    rel = xd_ref[...] - xs_ref[...]
    rf = _rfeat(ea, rel, off[...], a4[...], g20[...])
    pre = (_dot(ea.astype(jnp.bfloat16), w1e[...])
           + _dot(rf.astype(jnp.bfloat16), w1r[...])
           + _dot(hd, w1d[...]) + _dot(hs, w1s[...])) + b1[...]
    pk = pre[:, :HID]
    pv = pre[:, HID:]
    k = _dot(jnp.maximum(_ln(pk, gk[...], bek[...]), 0.0
                         ).astype(jnp.bfloat16), w2k[...]) + b2k[...]
    v = _dot(jnp.maximum(_ln(pv, gv[...], bev[...]), 0.0
                         ).astype(jnp.bfloat16), w2v[...]) + b2v[...]
    ewl = jnp.sum(rf * eww[...], -1, keepdims=True) + ewb[...]
    v = v * (1.0 / (1.0 + jnp.exp(-ewl)))
    ex = jnp.exp(_dot(qd * k, hs_m[...]) * _ISQ)
    ex_ref[...] = ex
    wv_ref[...] = _dot(ex, exh[...]) * v
    rel_ref[...] = rel


def _edge2_body(ea_ref, rel_ref, dd_ref, ss_ref,
                off, a4, g20, hs_m, p48, q48,
                w1e, w1r, w1d, w1s, b1,
                gk, bek, w2k, b2k, gv, bev, w2v, b2v,
                eww, ewb,
                ex_ref, wv_ref):
    ea = ea_ref[...]
    rel = rel_ref[...]
    hd = dd_ref[:, 0:HID]
    qd = dd_ref[:, HID:2 * HID].astype(jnp.float32)
    hs = ss_ref[...]
    rf = _rfeat(ea, rel, off[...], a4[...], g20[...])
    pre = (_dot(ea.astype(jnp.bfloat16), w1e[...])
           + _dot(rf.astype(jnp.bfloat16), w1r[...])
           + _dot(hd, w1d[...]) + _dot(hs, w1s[...])) + b1[...]
    pk = pre[:, :HID]
    pv = pre[:, HID:]
    k = _dot(jnp.maximum(_ln(pk, gk[...], bek[...]), 0.0
                         ).astype(jnp.bfloat16), w2k[...]) + b2k[...]
    v = _dot(jnp.maximum(_ln(pv, gv[...], bev[...]), 0.0
                         ).astype(jnp.bfloat16), w2v[...]) + b2v[...]
    ewl = jnp.sum(rf * eww[...], -1, keepdims=True) + ewb[...]
    v = v * (1.0 / (1.0 + jnp.exp(-ewl)))
    ex = jnp.exp(_dot(qd * k, hs_m[...]) * _ISQ)
    ex_ref[...] = ex
    wv_ref[...] = _dot(ex * v, p48[...]) * _dot(rel, q48[...])


def _node1_body(exs_ref, wvs_ref, h_ref, exh,
                w1a, w1b, b1, g1, be1, w2, b2, ho_ref):
    exb = _dot(exs_ref[...], exh[...])
    out1 = wvs_ref[...] / (exb + 1e-16)
    pre = _dot(out1, w1a[...]) + _dot(h_ref[...], w1b[...]) + b1[...]
    o = _dot(jnp.maximum(_ln(pre, g1[...], be1[...]), 0.0), w2[...]) + b2[...]
    ho_ref[...] = o + h_ref[...]


def _node2_body(exs_ref, wvs_ref, x_ref, m_ref, p48, m48, xo_ref):
    den = _dot(exs_ref[...], p48[...])
    o = wvs_ref[...] / (den + 1e-16)
    delta = _dot(o, m48[...])
    xo_ref[...] = x_ref[:, :8] + delta * m_ref[...]


_FULL = lambda r, c: pl.BlockSpec((r, c), lambda i: (0, 0))
_ROWB = lambda b, c: pl.BlockSpec((b, c), lambda i: (i, 0))


def _edge1_call(ea, dd, ss, xd, xs, consts, w):
    return pl.pallas_call(
        _edge1_body,
        grid=(E // BE,),
        in_specs=[
            _ROWB(BE, EFD), _ROWB(BE, 2 * HID), _ROWB(BE, HID),
            _ROWB(BE, 16), _ROWB(BE, 16),
            _FULL(1, NRG), _FULL(EFD, RFD), _FULL(NRG, RFD),
            _FULL(HID, HEADS), _FULL(HEADS, HID),
            _FULL(EFD, 2 * HID), _FULL(RFD, 2 * HID),
            _FULL(HID, 2 * HID), _FULL(HID, 2 * HID), _FULL(1, 2 * HID),
            _FULL(1, HID), _FULL(1, HID), _FULL(HID, HID), _FULL(1, HID),
            _FULL(1, HID), _FULL(1, HID), _FULL(HID, HID), _FULL(1, HID),
            _FULL(1, RFD), _FULL(1, 1),
        ],
        out_specs=[_ROWB(BE, HEADS), _ROWB(BE, HID), _ROWB(BE, 16)],
        out_shape=[
            jax.ShapeDtypeStruct((E, HEADS), jnp.float32),
            jax.ShapeDtypeStruct((E, HID), jnp.float32),
            jax.ShapeDtypeStruct((E, 16), jnp.float32),
        ],
    )(ea, dd, ss, xd, xs, *consts, *w)


def _edge2_call(ea, rel, dd, ss, consts, w):
    return pl.pallas_call(
        _edge2_body,
        grid=(E // BE,),
        in_specs=[
            _ROWB(BE, EFD), _ROWB(BE, 16), _ROWB(BE, 2 * HID), _ROWB(BE, HID),
            _FULL(1, NRG), _FULL(EFD, RFD), _FULL(NRG, RFD),
            _FULL(HID, HEADS), _FULL(HEADS, 48), _FULL(16, 48),
            _FULL(EFD, 2 * HID), _FULL(RFD, 2 * HID),
            _FULL(HID, 2 * HID), _FULL(HID, 2 * HID), _FULL(1, 2 * HID),
            _FULL(1, HID), _FULL(1, HID), _FULL(HID, HID), _FULL(1, HID),
            _FULL(1, HID), _FULL(1, HID), _FULL(HID, HEADS), _FULL(1, HEADS),
            _FULL(1, RFD), _FULL(1, 1),
        ],
        out_specs=[_ROWB(BE, HEADS), _ROWB(BE, 48)],
        out_shape=[
            jax.ShapeDtypeStruct((E, HEADS), jnp.float32),
            jax.ShapeDtypeStruct((E, 48), jnp.float32),
        ],
    )(ea, rel, dd, ss, *consts, *w)


def _split_w1(p, dout):
    w1 = p['W1']
    return (w1[0:EFD], w1[EFD:EFD + RFD],
            w1[EFD + RFD:EFD + RFD + HID], w1[EFD + RFD + HID:])


def _edge_weights(pk, pv, eww, ewb):
    ke, kr, kd, ks = _split_w1(pk, HID)
    ve, vr, vd, vs = _split_w1(pv, None)
    bf = jnp.bfloat16
    w1e = jnp.concatenate([ke, ve], 1).astype(bf)
    w1r = jnp.concatenate([kr, vr], 1).astype(bf)
    w1d = jnp.concatenate([kd, vd], 1).astype(bf)
    w1s = jnp.concatenate([ks, vs], 1).astype(bf)
    b1 = jnp.concatenate([pk['b1'], pv['b1']])[None, :]
    return [w1e, w1r, w1d, w1s, b1,
            pk['g'][None, :], pk['be'][None, :], pk['W2'].astype(bf),
            pk['b2'][None, :],
            pv['g'][None, :], pv['be'][None, :], pv['W2'].astype(bf),
            pv['b2'][None, :],
            eww.reshape(1, RFD), ewb.reshape(1, 1)]


def kernel(h, x, edge_attr, edge_index, mask_ligand, params):
    p = params
    src = edge_index[0]
    dst = edge_index[1]
    xpad = jnp.concatenate([x, jnp.zeros((N, 13), jnp.float32)], 1)

    off = jnp.asarray(_OFF)[None, :]
    a4 = jnp.asarray(_A4)
    g20 = jnp.asarray(_G20)
    hs_m = jnp.asarray(_HS)
    exh = jnp.asarray(_EXH)
    p48 = jnp.asarray(_P48)
    q48 = jnp.asarray(_Q48)
    m48 = jnp.asarray(_M48)

    zpad = jnp.zeros((_IDXP - _NCH, _CHW), jnp.int32)
    srcm = jnp.concatenate([src.reshape(_NCH, _CHW), zpad], 0)
    dstm = jnp.concatenate([dst.reshape(_NCH, _CHW), zpad], 0)

    q1 = _mlp128(h, p['x2h_hq'])
    dd1, xd = _sc_gather_multi(
        [jnp.concatenate([h, q1], 1).astype(jnp.bfloat16), xpad],
        [0, 0], dstm, srcm)
    ss1, xs = _sc_gather_multi(
        [h.astype(jnp.bfloat16), xpad], [1, 1], dstm, srcm)
    w1 = _edge_weights(p['x2h_hk'], p['x2h_hv'], p['x2h_ew_W'], p['x2h_ew_b'])
    ex1, wv1, rel = _edge1_call(edge_attr, dd1, ss1, xd, xs,
                                (off, a4, g20, hs_m, exh), w1)
    exs1, wvs1 = _sc_scatter_split(ex1, wv1, dstm, HID)

    po = p['x2h_out']
    hout = pl.pallas_call(
        _node1_body,
        grid=(N // BN,),
        in_specs=[
            _ROWB(BN, HEADS), _ROWB(BN, HID), _ROWB(BN, HID),
            _FULL(HEADS, HID),
            _FULL(HID, HID), _FULL(HID, HID), _FULL(1, HID),
            _FULL(1, HID), _FULL(1, HID), _FULL(HID, HID), _FULL(1, HID),
        ],
        out_specs=_ROWB(BN, HID),
        out_shape=jax.ShapeDtypeStruct((N, HID), jnp.float32),
    )(exs1, wvs1, h, exh,
      po['W1'][:HID], po['W1'][HID:], po['b1'][None, :],
      po['g'][None, :], po['be'][None, :], po['W2'], po['b2'][None, :])

    q2 = _mlp128(hout, p['h2x_xq'])
    dd2, ss2 = _sc_gather_multi(
        [jnp.concatenate([hout, q2], 1).astype(jnp.bfloat16),
         hout.astype(jnp.bfloat16)], [0, 1], dstm, srcm)
    w2 = _edge_weights(p['h2x_xk'], p['h2x_xv'], p['h2x_ew_W'], p['h2x_ew_b'])
    ex2, wv2 = _edge2_call(edge_attr, rel, dd2, ss2,
                           (off, a4, g20, hs_m, p48, q48), w2)
    exs2, wvs2 = _sc_scatter_split(ex2, wv2, dstm, 48)

    x8 = pl.pallas_call(
        _node2_body,
        grid=(N // BN,),
        in_specs=[
            _ROWB(BN, HEADS), _ROWB(BN, 48), _ROWB(BN, 16), _ROWB(BN, 8),
            _FULL(HEADS, 48), _FULL(48, 8),
        ],
        out_specs=_ROWB(BN, 8),
        out_shape=jax.ShapeDtypeStruct((N, 8), jnp.float32),
    )(exs2, wvs2, xpad,
      jnp.broadcast_to(mask_ligand[:, None], (N, 8)), p48, m48)

    return (hout, x8[:, :3])


# speedup vs baseline: 1.0428x; 1.0428x over previous
import functools
import numpy as np
import jax
import jax.numpy as jnp
from jax import lax
from jax.experimental import pallas as pl
from jax.experimental.pallas import tpu as pltpu
from jax.experimental.pallas import tpu_sc as plsc

N = 10000
E = 160000
HID = 128
HEADS = 16
DH = 8
NRG = 20
EFD = 4
RFD = NRG * 4

BE = 2000
BN = 2000

_OFF = np.linspace(0.0, 10.0, NRG).astype(np.float32)
_COEFF = np.float32(-0.5 / (_OFF[1] - _OFF[0]) ** 2)
_A4 = np.zeros((EFD, RFD), np.float32)
_G20 = np.zeros((NRG, RFD), np.float32)
for _a in range(EFD):
    for _g in range(NRG):
        _A4[_a, _a * NRG + _g] = 1.0
        _G20[_g, _a * NRG + _g] = 1.0
_HS = np.zeros((HID, HEADS), np.float32)
_EXH = np.zeros((HEADS, HID), np.float32)
for _h in range(HEADS):
    for _j in range(DH):
        _HS[_h * DH + _j, _h] = 1.0
        _EXH[_h, _h * DH + _j] = 1.0
_P48 = np.zeros((HEADS, 48), np.float32)
_Q48 = np.zeros((16, 48), np.float32)
_M48 = np.zeros((48, 8), np.float32)
for _h in range(HEADS):
    for _c in range(3):
        _P48[_h, 3 * _h + _c] = 1.0
        _Q48[_c, 3 * _h + _c] = 1.0
        _M48[3 * _h + _c, _c] = 1.0 / HEADS
_ISQ = np.float32(1.0 / np.sqrt(DH))


_CHW = 128
_NCH = E // _CHW
_NW = 32
_NB = 40
_IDXP = 1264
_EOUT = _NCH * _CHW + _CHW

_SC_MESH = dict(core_axis_name="c", subcore_axis_name="s")
_SC_PARAMS = pltpu.CompilerParams(use_tc_tiling_on_sc=False)


def _worker_start(wid):
    return 39 * wid + jnp.minimum(wid, 2)


def _sc_gather_multi(tables, sels, dstm, srcm):
    nt = len(tables)
    ds = [int(t.shape[1]) for t in tables]
    dts = [t.dtype for t in tables]

    @functools.partial(
        pl.kernel,
        out_type=tuple(jax.ShapeDtypeStruct((_EOUT, d), dt)
                       for d, dt in zip(ds, dts)),
        mesh=plsc.VectorSubcoreMesh(**_SC_MESH),
        compiler_params=_SC_PARAMS,
        scratch_types=(
            [pltpu.VMEM((_NB, _CHW), jnp.int32)] * 2
            + [pltpu.VMEM((_CHW, d), dt)
               for d, dt in zip(ds, dts) for _ in (0, 1)]
            + [pltpu.SemaphoreType.DMA] * (4 * nt)
        ),
    )
    def k(*refs):
        tabs = refs[0:nt]
        outs = refs[nt + 2:2 * nt + 2]
        idxa = refs[2 * nt + 2:2 * nt + 4]
        rows = [refs[2 * nt + 4 + 2 * t:2 * nt + 6 + 2 * t]
                for t in range(nt)]
        gs = [refs[4 * nt + 4 + 2 * t:4 * nt + 6 + 2 * t]
              for t in range(nt)]
        osm = [refs[6 * nt + 4 + 2 * t:6 * nt + 6 + 2 * t]
               for t in range(nt)]
        wid = lax.axis_index("s") * 2 + lax.axis_index("c")
        start = _worker_start(wid)
        pltpu.sync_copy(refs[nt].at[pl.ds(start, _NB)], idxa[0])
        pltpu.sync_copy(refs[nt + 1].at[pl.ds(start, _NB)], idxa[1])
        for t in range(nt):
            pltpu.async_copy(tabs[t].at[idxa[sels[t]].at[0]],
                             rows[t][0], gs[t][0])

        def pair(jj, carry):
            for bb in (0, 1):
                j = jj * 2 + bb
                ob = 1 - bb
                for t in range(nt):
                    pltpu.make_async_copy(
                        tabs[t].at[idxa[sels[t]].at[j]],
                        rows[t][bb], gs[t][bb]).wait()
                    pltpu.async_copy(
                        rows[t][bb],
                        outs[t].at[pl.ds((start + j) * _CHW, _CHW)],
                        osm[t][bb])

                @pl.when(j < _NB - 1)
                def _():
                    @pl.when(j >= 1)
                    def _():
                        for t in range(nt):
                            pltpu.make_async_copy(
                                rows[t][ob], outs[t].at[pl.ds(0, _CHW)],
                                osm[t][ob]).wait()
                    for t in range(nt):
                        pltpu.async_copy(
                            tabs[t].at[idxa[sels[t]].at[j + 1]],
                            rows[t][ob], gs[t][ob])
            return carry

        lax.fori_loop(0, _NB // 2, pair, 0)
        for bb in (0, 1):
            for t in range(nt):
                pltpu.make_async_copy(
                    rows[t][bb], outs[t].at[pl.ds(0, _CHW)],
                    osm[t][bb]).wait()

    return k(*tables, dstm, srcm)


_HN = N // 2
_TB = _HN + 8
_NB2 = 79


def _sc_scatter_split(exw, wv, idxm, d):
    @functools.partial(
        pl.kernel,
        out_type=(jax.ShapeDtypeStruct((N, 16), jnp.float32),
                  jax.ShapeDtypeStruct((N, d), jnp.float32)),
        mesh=plsc.VectorSubcoreMesh(**_SC_MESH),
        compiler_params=_SC_PARAMS,
        scratch_types=[
            pltpu.VMEM((_NB2, _CHW), jnp.int32),
            pltpu.VMEM((_CHW,), jnp.int32),
            pltpu.VMEM((_CHW, 16), jnp.float32),
            pltpu.VMEM((_CHW, 16), jnp.float32),
            pltpu.VMEM((_CHW, d), jnp.float32),
            pltpu.VMEM((_CHW, d), jnp.float32),
            pltpu.VMEM_SHARED((_TB, 16), jnp.float32),
            pltpu.VMEM_SHARED((_TB, d), jnp.float32),
            pltpu.SemaphoreType.DMA,
            pltpu.SemaphoreType.DMA,
        ],
    )
    def k(ex_hbm, wv_hbm, idxm_hbm, zex_hbm, zwv_hbm, oex_hbm, owv_hbm,
          idxa, idxc, exb0, exb1, wvb0, wvb1, tex, twv, ls0, ls1):
        cid = lax.axis_index("c")
        sid = lax.axis_index("s")
        base = cid * _HN
        c0 = sid * _NB2
        nch = jnp.minimum(_NB2, _NCH - c0)
        exb = (exb0, exb1)
        wvb = (wvb0, wvb1)
        ls = (ls0, ls1)

        def load(j, bb):
            pltpu.async_copy(
                ex_hbm.at[pl.ds((c0 + j) * _CHW, _CHW)], exb[bb], ls[bb])
            pltpu.async_copy(
                wv_hbm.at[pl.ds((c0 + j) * _CHW, _CHW)], wvb[bb], ls[bb])

        def load_wait(bb):
            pltpu.make_async_copy(
                ex_hbm.at[pl.ds(0, _CHW)], exb[bb], ls[bb]).wait()
            pltpu.make_async_copy(
                wv_hbm.at[pl.ds(0, _CHW)], wvb[bb], ls[bb]).wait()

        pltpu.sync_copy(idxm_hbm.at[pl.ds(c0, _NB2)], idxa)
        load(0, 0)

        @pl.when(sid == 0)
        def _init():
            pltpu.sync_copy(zex_hbm, tex)
            pltpu.sync_copy(zwv_hbm, twv)

        plsc.subcore_barrier()

        def pair(jj, carry):
            for bb in (0, 1):
                j = jj * 2 + bb

                @pl.when(j < nch)
                def _():
                    load_wait(bb)

                    @pl.when(j + 1 < nch)
                    def _():
                        load(j + 1, 1 - bb)

                    for l in range(_CHW // 16):
                        iv = idxa[j, pl.ds(l * 16, 16)]
                        inr = (iv >= base) & (iv < base + _HN)
                        idxc[pl.ds(l * 16, 16)] = jnp.where(
                            inr, iv - base, _HN)
                    pltpu.sync_copy(exb[bb], tex.at[idxc], add=True)
                    pltpu.sync_copy(wvb[bb], twv.at[idxc], add=True)
            return carry

        lax.fori_loop(0, (_NB2 + 1) // 2, pair, 0)
        plsc.subcore_barrier()

        @pl.when(sid == 0)
        def _dump():
            pltpu.sync_copy(tex.at[pl.ds(0, _HN)],
                            oex_hbm.at[pl.ds(base, _HN)])
            pltpu.sync_copy(twv.at[pl.ds(0, _HN)],
                            owv_hbm.at[pl.ds(base, _HN)])

    zex = jnp.zeros((_TB, 16), jnp.float32)
    zwv = jnp.zeros((_TB, d), jnp.float32)
    return k(exw, wv, idxm, zex, zwv)


def _ln(v, g, b):
    mu = jnp.mean(v, -1, keepdims=True)
    var = jnp.mean((v - mu) ** 2, -1, keepdims=True)
    return (v - mu) / jnp.sqrt(var + 1e-5) * g + b


def _dot(a, b):
    return jnp.dot(a, b, preferred_element_type=jnp.float32)


def _mlp128_body(h_ref, w1, b1, g1, be1, w2, b2, o_ref):
    v = _dot(h_ref[...], w1[...]) + b1[...]
    v = jnp.maximum(_ln(v, g1[...], be1[...]), 0.0)
    o_ref[...] = _dot(v, w2[...]) + b2[...]


def _mlp128(hin, p):
    return pl.pallas_call(
        _mlp128_body,
        grid=(N // BN,),
        in_specs=[
            pl.BlockSpec((BN, HID), lambda i: (i, 0)),
            pl.BlockSpec((HID, HID), lambda i: (0, 0)),
            pl.BlockSpec((1, HID), lambda i: (0, 0)),
            pl.BlockSpec((1, HID), lambda i: (0, 0)),
            pl.BlockSpec((1, HID), lambda i: (0, 0)),
            pl.BlockSpec((HID, HID), lambda i: (0, 0)),
            pl.BlockSpec((1, HID), lambda i: (0, 0)),
        ],
        out_specs=pl.BlockSpec((BN, HID), lambda i: (i, 0)),
        out_shape=jax.ShapeDtypeStruct((N, HID), jnp.float32),
    )(hin, p['W1'], p['b1'][None, :], p['g'][None, :], p['be'][None, :],
      p['W2'], p['b2'][None, :])


def _rfeat(ea, rel, off, a4, g20):
    d2 = jnp.sum(rel * rel, -1, keepdims=True)
    dist = jnp.sqrt(d2)
    sm = jnp.exp(_COEFF * (dist - off) ** 2)
    return _dot(ea, a4) * _dot(sm, g20)


def _edge1_body(ea_ref, dd_ref, ss_ref, xd_ref, xs_ref,
                off, a4, g20, hs_m, exh,
                w1e, w1r, w1d, w1s, b1,
                gk, bek, w2k, b2k, gv, bev, w2v, b2v,
                eww, ewb,
                ex_ref, wv_ref, rel_ref):
    ea = ea_ref[...]
    hd = dd_ref[:, 0:HID]
    qd = dd_ref[:, HID:2 * HID].astype(jnp.float32)
    hs = ss_ref[...]
    rel = xd_ref[...] - xs_ref[...]
    rf = _rfeat(ea, rel, off[...], a4[...], g20[...])
    hdf = hd.astype(jnp.float32)
    hsf = hs.astype(jnp.float32)
    pre = (_dot(ea, w1e[...]) + _dot(rf, w1r[...])
           + _dot(hdf, w1d[...]) + _dot(hsf, w1s[...])) + b1[...]
    pk = pre[:, :HID]
    pv = pre[:, HID:]
    k = _dot(jnp.maximum(_ln(pk, gk[...], bek[...]), 0.0), w2k[...]) + b2k[...]
    v = _dot(jnp.maximum(_ln(pv, gv[...], bev[...]), 0.0), w2v[...]) + b2v[...]
    ewl = jnp.sum(rf * eww[...], -1, keepdims=True) + ewb[...]
    v = v * (1.0 / (1.0 + jnp.exp(-ewl)))
    ex = jnp.exp(_dot(qd * k, hs_m[...]) * _ISQ)
    ex_ref[...] = ex
    wv_ref[...] = _dot(ex, exh[...]) * v
    rel_ref[...] = rel


def _edge2_body(ea_ref, rel_ref, dd_ref, ss_ref,
                off, a4, g20, hs_m, p48, q48,
                w1e, w1r, w1d, w1s, b1,
                gk, bek, w2k, b2k, gv, bev, w2v, b2v,
                eww, ewb,
                ex_ref, wv_ref):
    ea = ea_ref[...]
    rel = rel_ref[...]
    hd = dd_ref[:, 0:HID]
    qd = dd_ref[:, HID:2 * HID].astype(jnp.float32)
    hs = ss_ref[...]
    rf = _rfeat(ea, rel, off[...], a4[...], g20[...])
    hdf = hd.astype(jnp.float32)
    hsf = hs.astype(jnp.float32)
    pre = (_dot(ea, w1e[...]) + _dot(rf, w1r[...])
           + _dot(hdf, w1d[...]) + _dot(hsf, w1s[...])) + b1[...]
    pk = pre[:, :HID]
    pv = pre[:, HID:]
    k = _dot(jnp.maximum(_ln(pk, gk[...], bek[...]), 0.0), w2k[...]) + b2k[...]
    v = _dot(jnp.maximum(_ln(pv, gv[...], bev[...]), 0.0), w2v[...]) + b2v[...]
    ewl = jnp.sum(rf * eww[...], -1, keepdims=True) + ewb[...]
    v = v * (1.0 / (1.0 + jnp.exp(-ewl)))
    ex = jnp.exp(_dot(qd * k, hs_m[...]) * _ISQ)
    ex_ref[...] = ex
    wv_ref[...] = _dot(ex * v, p48[...]) * _dot(rel, q48[...])


def _node1_body(exs_ref, wvs_ref, h_ref, exh,
                w1a, w1b, b1, g1, be1, w2, b2, ho_ref):
    exb = _dot(exs_ref[...], exh[...])
    out1 = wvs_ref[...] / (exb + 1e-16)
    pre = _dot(out1, w1a[...]) + _dot(h_ref[...], w1b[...]) + b1[...]
    o = _dot(jnp.maximum(_ln(pre, g1[...], be1[...]), 0.0), w2[...]) + b2[...]
    ho_ref[...] = o + h_ref[...]


def _node2_body(exs_ref, wvs_ref, x_ref, m_ref, p48, m48, xo_ref):
    den = _dot(exs_ref[...], p48[...])
    o = wvs_ref[...] / (den + 1e-16)
    delta = _dot(o, m48[...])
    xo_ref[...] = x_ref[:, :8] + delta * m_ref[...]


_FULL = lambda r, c: pl.BlockSpec((r, c), lambda i: (0, 0))
_ROWB = lambda b, c: pl.BlockSpec((b, c), lambda i: (i, 0))


def _edge1_call(ea, dd, ss, xd, xs, consts, w):
    return pl.pallas_call(
        _edge1_body,
        grid=(E // BE,),
        in_specs=[
            _ROWB(BE, EFD), _ROWB(BE, 2 * HID), _ROWB(BE, HID),
            _ROWB(BE, 16), _ROWB(BE, 16),
            _FULL(1, NRG), _FULL(EFD, RFD), _FULL(NRG, RFD),
            _FULL(HID, HEADS), _FULL(HEADS, HID),
            _FULL(EFD, 2 * HID), _FULL(RFD, 2 * HID),
            _FULL(HID, 2 * HID), _FULL(HID, 2 * HID), _FULL(1, 2 * HID),
            _FULL(1, HID), _FULL(1, HID), _FULL(HID, HID), _FULL(1, HID),
            _FULL(1, HID), _FULL(1, HID), _FULL(HID, HID), _FULL(1, HID),
            _FULL(1, RFD), _FULL(1, 1),
        ],
        out_specs=[_ROWB(BE, HEADS), _ROWB(BE, HID), _ROWB(BE, 16)],
        out_shape=[
            jax.ShapeDtypeStruct((E, HEADS), jnp.float32),
            jax.ShapeDtypeStruct((E, HID), jnp.float32),
            jax.ShapeDtypeStruct((E, 16), jnp.float32),
        ],
    )(ea, dd, ss, xd, xs, *consts, *w)


def _edge2_call(ea, rel, dd, ss, consts, w):
    return pl.pallas_call(
        _edge2_body,
        grid=(E // BE,),
        in_specs=[
            _ROWB(BE, EFD), _ROWB(BE, 16), _ROWB(BE, 2 * HID), _ROWB(BE, HID),
            _FULL(1, NRG), _FULL(EFD, RFD), _FULL(NRG, RFD),
            _FULL(HID, HEADS), _FULL(HEADS, 48), _FULL(16, 48),
            _FULL(EFD, 2 * HID), _FULL(RFD, 2 * HID),
            _FULL(HID, 2 * HID), _FULL(HID, 2 * HID), _FULL(1, 2 * HID),
            _FULL(1, HID), _FULL(1, HID), _FULL(HID, HID), _FULL(1, HID),
            _FULL(1, HID), _FULL(1, HID), _FULL(HID, HEADS), _FULL(1, HEADS),
            _FULL(1, RFD), _FULL(1, 1),
        ],
        out_specs=[_ROWB(BE, HEADS), _ROWB(BE, 48)],
        out_shape=[
            jax.ShapeDtypeStruct((E, HEADS), jnp.float32),
            jax.ShapeDtypeStruct((E, 48), jnp.float32),
        ],
    )(ea, rel, dd, ss, *consts, *w)


def _split_w1(p, dout):
    w1 = p['W1']
    return (w1[0:EFD], w1[EFD:EFD + RFD],
            w1[EFD + RFD:EFD + RFD + HID], w1[EFD + RFD + HID:])


def _edge_weights(pk, pv, eww, ewb):
    ke, kr, kd, ks = _split_w1(pk, HID)
    ve, vr, vd, vs = _split_w1(pv, None)
    w1e = jnp.concatenate([ke, ve], 1)
    w1r = jnp.concatenate([kr, vr], 1)
    w1d = jnp.concatenate([kd, vd], 1)
    w1s = jnp.concatenate([ks, vs], 1)
    b1 = jnp.concatenate([pk['b1'], pv['b1']])[None, :]
    return [w1e, w1r, w1d, w1s, b1,
            pk['g'][None, :], pk['be'][None, :], pk['W2'], pk['b2'][None, :],
            pv['g'][None, :], pv['be'][None, :], pv['W2'], pv['b2'][None, :],
            eww.reshape(1, RFD), ewb.reshape(1, 1)]


def kernel(h, x, edge_attr, edge_index, mask_ligand, params):
    p = params
    src = edge_index[0]
    dst = edge_index[1]
    xpad = jnp.concatenate([x, jnp.zeros((N, 13), jnp.float32)], 1)

    off = jnp.asarray(_OFF)[None, :]
    a4 = jnp.asarray(_A4)
    g20 = jnp.asarray(_G20)
    hs_m = jnp.asarray(_HS)
    exh = jnp.asarray(_EXH)
    p48 = jnp.asarray(_P48)
    q48 = jnp.asarray(_Q48)
    m48 = jnp.asarray(_M48)

    zpad = jnp.zeros((_IDXP - _NCH, _CHW), jnp.int32)
    srcm = jnp.concatenate([src.reshape(_NCH, _CHW), zpad], 0)
    dstm = jnp.concatenate([dst.reshape(_NCH, _CHW), zpad], 0)

    q1 = _mlp128(h, p['x2h_hq'])
    dd1, xd = _sc_gather_multi(
        [jnp.concatenate([h, q1], 1).astype(jnp.bfloat16), xpad],
        [0, 0], dstm, srcm)
    ss1, xs = _sc_gather_multi(
        [h.astype(jnp.bfloat16), xpad], [1, 1], dstm, srcm)
    w1 = _edge_weights(p['x2h_hk'], p['x2h_hv'], p['x2h_ew_W'], p['x2h_ew_b'])
    ex1, wv1, rel = _edge1_call(edge_attr, dd1, ss1, xd, xs,
                                (off, a4, g20, hs_m, exh), w1)
    exs1, wvs1 = _sc_scatter_split(ex1, wv1, dstm, HID)

    po = p['x2h_out']
    hout = pl.pallas_call(
        _node1_body,
        grid=(N // BN,),
        in_specs=[
            _ROWB(BN, HEADS), _ROWB(BN, HID), _ROWB(BN, HID),
            _FULL(HEADS, HID),
            _FULL(HID, HID), _FULL(HID, HID), _FULL(1, HID),
            _FULL(1, HID), _FULL(1, HID), _FULL(HID, HID), _FULL(1, HID),
        ],
        out_specs=_ROWB(BN, HID),
        out_shape=jax.ShapeDtypeStruct((N, HID), jnp.float32),
    )(exs1, wvs1, h, exh,
      po['W1'][:HID], po['W1'][HID:], po['b1'][None, :],
      po['g'][None, :], po['be'][None, :], po['W2'], po['b2'][None, :])

    q2 = _mlp128(hout, p['h2x_xq'])
    dd2, ss2 = _sc_gather_multi(
        [jnp.concatenate([hout, q2], 1).astype(jnp.bfloat16),
         hout.astype(jnp.bfloat16)], [0, 1], dstm, srcm)
    w2 = _edge_weights(p['h2x_xk'], p['h2x_xv'], p['h2x_ew_W'], p['h2x_ew_b'])
    ex2, wv2 = _edge2_call(edge_attr, rel, dd2, ss2,
                           (off, a4, g20, hs_m, p48, q48), w2)
    exs2, wvs2 = _sc_scatter_split(ex2, wv2, dstm, 48)

    x8 = pl.pallas_call(
        _node2_body,
        grid=(N // BN,),
        in_specs=[
            _ROWB(BN, HEADS), _ROWB(BN, 48), _ROWB(BN, 16), _ROWB(BN, 8),
            _FULL(HEADS, 48), _FULL(48, 8),
        ],
        out_specs=_ROWB(BN, 8),
        out_shape=jax.ShapeDtypeStruct((N, 8), jnp.float32),
    )(exs2, wvs2, xpad,
      jnp.broadcast_to(mask_ligand[:, None], (N, 8)), p48, m48)

    return (hout, x8[:, :3])


# speedup vs baseline: 1.1219x; 1.0759x over previous
import functools
import numpy as np
import jax
import jax.numpy as jnp
from jax import lax
from jax.experimental import pallas as pl
from jax.experimental.pallas import tpu as pltpu
from jax.experimental.pallas import tpu_sc as plsc

N = 10000
E = 160000
HID = 128
HEADS = 16
DH = 8
NRG = 20
EFD = 4
RFD = NRG * 4

BE = 2000
BN = 2000

_OFF = np.linspace(0.0, 10.0, NRG).astype(np.float32)
_COEFF = np.float32(-0.5 / (_OFF[1] - _OFF[0]) ** 2)
_A4 = np.zeros((EFD, RFD), np.float32)
_G20 = np.zeros((NRG, RFD), np.float32)
for _a in range(EFD):
    for _g in range(NRG):
        _A4[_a, _a * NRG + _g] = 1.0
        _G20[_g, _a * NRG + _g] = 1.0
_HS = np.zeros((HID, HEADS), np.float32)
_EXH = np.zeros((HEADS, HID), np.float32)
for _h in range(HEADS):
    for _j in range(DH):
        _HS[_h * DH + _j, _h] = 1.0
        _EXH[_h, _h * DH + _j] = 1.0
_P48 = np.zeros((HEADS, 48), np.float32)
_Q48 = np.zeros((16, 48), np.float32)
_M48 = np.zeros((48, 8), np.float32)
for _h in range(HEADS):
    for _c in range(3):
        _P48[_h, 3 * _h + _c] = 1.0
        _Q48[_c, 3 * _h + _c] = 1.0
        _M48[3 * _h + _c, _c] = 1.0 / HEADS
_ISQ = np.float32(1.0 / np.sqrt(DH))


_CHW = 128
_NCH = E // _CHW
_NW = 32

_SC_MESH = dict(core_axis_name="c", subcore_axis_name="s")
_SC_PARAMS = pltpu.CompilerParams(use_tc_tiling_on_sc=False)


def _sc_gather(table, idxm, d):
    @functools.partial(
        pl.kernel,
        out_type=jax.ShapeDtypeStruct((E, d), jnp.float32),
        mesh=plsc.VectorSubcoreMesh(**_SC_MESH),
        compiler_params=_SC_PARAMS,
        scratch_types=[
            pltpu.VMEM((_CHW,), jnp.int32),
            pltpu.VMEM((_CHW, d), jnp.float32),
            pltpu.SemaphoreType.DMA,
        ],
    )
    def k(table_hbm, idxm_hbm, out_hbm, idx_v, rows_v, sem):
        wid = lax.axis_index("s") * 2 + lax.axis_index("c")
        nch = (_NCH - wid + (_NW - 1)) // _NW

        def body(t, carry):
            ch = wid + t * _NW
            pltpu.sync_copy(idxm_hbm.at[ch], idx_v)
            pltpu.async_copy(table_hbm.at[idx_v], rows_v, sem).wait()
            pltpu.sync_copy(rows_v, out_hbm.at[pl.ds(ch * _CHW, _CHW)])
            return carry

        lax.fori_loop(0, nch, body, 0)

    return k(table, idxm)


def _sc_scatter(exw, wv, idxm, d):
    @functools.partial(
        pl.kernel,
        out_type=(jax.ShapeDtypeStruct((2, N, 16), jnp.float32),
                  jax.ShapeDtypeStruct((2, N, d), jnp.float32)),
        mesh=plsc.VectorSubcoreMesh(**_SC_MESH),
        compiler_params=_SC_PARAMS,
        scratch_types=[
            pltpu.VMEM((_CHW,), jnp.int32),
            pltpu.VMEM((_CHW, 16), jnp.float32),
            pltpu.VMEM((_CHW, d), jnp.float32),
            pltpu.VMEM_SHARED((N, 16), jnp.float32),
            pltpu.VMEM_SHARED((N, d), jnp.float32),
        ],
    )
    def k(ex_hbm, wv_hbm, idxm_hbm, zex_hbm, zwv_hbm, oex_hbm, owv_hbm,
          idx_v, exb, wvb, tex, twv):
        cid = lax.axis_index("c")
        sid = lax.axis_index("s")
        wid = sid * 2 + cid

        @pl.when(sid == 0)
        def _init():
            pltpu.sync_copy(zex_hbm, tex)
            pltpu.sync_copy(zwv_hbm, twv)

        plsc.subcore_barrier()
        nch = (_NCH - wid + (_NW - 1)) // _NW

        def body(t, carry):
            ch = wid + t * _NW
            pltpu.sync_copy(idxm_hbm.at[ch], idx_v)
            pltpu.sync_copy(ex_hbm.at[pl.ds(ch * _CHW, _CHW)], exb)
            pltpu.sync_copy(wv_hbm.at[pl.ds(ch * _CHW, _CHW)], wvb)
            pltpu.sync_copy(exb, tex.at[idx_v], add=True)
            pltpu.sync_copy(wvb, twv.at[idx_v], add=True)
            return carry

        lax.fori_loop(0, nch, body, 0)
        plsc.subcore_barrier()

        @pl.when(sid == 0)
        def _dump():
            pltpu.sync_copy(tex, oex_hbm.at[cid])
            pltpu.sync_copy(twv, owv_hbm.at[cid])

    zex = jnp.zeros((N, 16), jnp.float32)
    zwv = jnp.zeros((N, d), jnp.float32)
    return k(exw, wv, idxm, zex, zwv)


def _ln(v, g, b):
    mu = jnp.mean(v, -1, keepdims=True)
    var = jnp.mean((v - mu) ** 2, -1, keepdims=True)
    return (v - mu) / jnp.sqrt(var + 1e-5) * g + b


def _dot(a, b):
    return jnp.dot(a, b, preferred_element_type=jnp.float32)


def _mlp128_body(h_ref, w1, b1, g1, be1, w2, b2, o_ref):
    v = _dot(h_ref[...], w1[...]) + b1[...]
    v = jnp.maximum(_ln(v, g1[...], be1[...]), 0.0)
    o_ref[...] = _dot(v, w2[...]) + b2[...]


def _mlp128(hin, p):
    return pl.pallas_call(
        _mlp128_body,
        grid=(N // BN,),
        in_specs=[
            pl.BlockSpec((BN, HID), lambda i: (i, 0)),
            pl.BlockSpec((HID, HID), lambda i: (0, 0)),
            pl.BlockSpec((1, HID), lambda i: (0, 0)),
            pl.BlockSpec((1, HID), lambda i: (0, 0)),
            pl.BlockSpec((1, HID), lambda i: (0, 0)),
            pl.BlockSpec((HID, HID), lambda i: (0, 0)),
            pl.BlockSpec((1, HID), lambda i: (0, 0)),
        ],
        out_specs=pl.BlockSpec((BN, HID), lambda i: (i, 0)),
        out_shape=jax.ShapeDtypeStruct((N, HID), jnp.float32),
    )(hin, p['W1'], p['b1'][None, :], p['g'][None, :], p['be'][None, :],
      p['W2'], p['b2'][None, :])


def _rfeat(ea, rel, off, a4, g20):
    d2 = jnp.sum(rel * rel, -1, keepdims=True)
    dist = jnp.sqrt(d2)
    sm = jnp.exp(_COEFF * (dist - off) ** 2)
    return _dot(ea, a4) * _dot(sm, g20)


def _edge1_body(ea_ref, dd_ref, ss_ref,
                off, a4, g20, hs_m, exh,
                w1e, w1r, w1d, w1s, b1,
                gk, bek, w2k, b2k, gv, bev, w2v, b2v,
                eww, ewb,
                ex_ref, wv_ref, rel_ref):
    ea = ea_ref[...]
    hd = dd_ref[:, 0:HID]
    qd = dd_ref[:, HID:2 * HID]
    xd = dd_ref[:, 2 * HID:2 * HID + 16]
    hs = ss_ref[:, 0:HID]
    xs = ss_ref[:, HID:HID + 16]
    rel = xd - xs
    rf = _rfeat(ea, rel, off[...], a4[...], g20[...])
    pre = (_dot(ea, w1e[...]) + _dot(rf, w1r[...])
           + _dot(hd, w1d[...]) + _dot(hs, w1s[...])) + b1[...]
    pk = pre[:, :HID]
    pv = pre[:, HID:]
    k = _dot(jnp.maximum(_ln(pk, gk[...], bek[...]), 0.0), w2k[...]) + b2k[...]
    v = _dot(jnp.maximum(_ln(pv, gv[...], bev[...]), 0.0), w2v[...]) + b2v[...]
    ewl = jnp.sum(rf * eww[...], -1, keepdims=True) + ewb[...]
    v = v * (1.0 / (1.0 + jnp.exp(-ewl)))
    ex = jnp.exp(_dot(qd * k, hs_m[...]) * _ISQ)
    ex_ref[...] = ex
    wv_ref[...] = _dot(ex, exh[...]) * v
    rel_ref[...] = rel


def _edge2_body(ea_ref, rel_ref, dd_ref, ss_ref,
                off, a4, g20, hs_m, p48, q48,
                w1e, w1r, w1d, w1s, b1,
                gk, bek, w2k, b2k, gv, bev, w2v, b2v,
                eww, ewb,
                ex_ref, wv_ref):
    ea = ea_ref[...]
    rel = rel_ref[...]
    hd = dd_ref[:, 0:HID]
    qd = dd_ref[:, HID:2 * HID]
    hs = ss_ref[...]
    rf = _rfeat(ea, rel, off[...], a4[...], g20[...])
    pre = (_dot(ea, w1e[...]) + _dot(rf, w1r[...])
           + _dot(hd, w1d[...]) + _dot(hs, w1s[...])) + b1[...]
    pk = pre[:, :HID]
    pv = pre[:, HID:]
    k = _dot(jnp.maximum(_ln(pk, gk[...], bek[...]), 0.0), w2k[...]) + b2k[...]
    v = _dot(jnp.maximum(_ln(pv, gv[...], bev[...]), 0.0), w2v[...]) + b2v[...]
    ewl = jnp.sum(rf * eww[...], -1, keepdims=True) + ewb[...]
    v = v * (1.0 / (1.0 + jnp.exp(-ewl)))
    ex = jnp.exp(_dot(qd * k, hs_m[...]) * _ISQ)
    ex_ref[...] = ex
    wv_ref[...] = _dot(ex * v, p48[...]) * _dot(rel, q48[...])


def _node1_body(ex0_ref, ex1_ref, wv0_ref, wv1_ref, h_ref, exh,
                w1a, w1b, b1, g1, be1, w2, b2, ho_ref):
    exb = _dot(ex0_ref[...] + ex1_ref[...], exh[...])
    out1 = (wv0_ref[...] + wv1_ref[...]) / (exb + 1e-16)
    pre = _dot(out1, w1a[...]) + _dot(h_ref[...], w1b[...]) + b1[...]
    o = _dot(jnp.maximum(_ln(pre, g1[...], be1[...]), 0.0), w2[...]) + b2[...]
    ho_ref[...] = o + h_ref[...]


def _node2_body(ex0_ref, ex1_ref, wv0_ref, wv1_ref, x_ref, m_ref,
                p48, m48, xo_ref):
    den = _dot(ex0_ref[...] + ex1_ref[...], p48[...])
    o = (wv0_ref[...] + wv1_ref[...]) / (den + 1e-16)
    delta = _dot(o, m48[...])
    xo_ref[...] = x_ref[:, :8] + delta * m_ref[...]


_FULL = lambda r, c: pl.BlockSpec((r, c), lambda i: (0, 0))
_ROWB = lambda b, c: pl.BlockSpec((b, c), lambda i: (i, 0))


def _edge1_call(ea, dd, ss, consts, w):
    return pl.pallas_call(
        _edge1_body,
        grid=(E // BE,),
        in_specs=[
            _ROWB(BE, EFD), _ROWB(BE, 2 * HID + 16), _ROWB(BE, HID + 16),
            _FULL(1, NRG), _FULL(EFD, RFD), _FULL(NRG, RFD),
            _FULL(HID, HEADS), _FULL(HEADS, HID),
            _FULL(EFD, 2 * HID), _FULL(RFD, 2 * HID),
            _FULL(HID, 2 * HID), _FULL(HID, 2 * HID), _FULL(1, 2 * HID),
            _FULL(1, HID), _FULL(1, HID), _FULL(HID, HID), _FULL(1, HID),
            _FULL(1, HID), _FULL(1, HID), _FULL(HID, HID), _FULL(1, HID),
            _FULL(1, RFD), _FULL(1, 1),
        ],
        out_specs=[_ROWB(BE, HEADS), _ROWB(BE, HID), _ROWB(BE, 16)],
        out_shape=[
            jax.ShapeDtypeStruct((E, HEADS), jnp.float32),
            jax.ShapeDtypeStruct((E, HID), jnp.float32),
            jax.ShapeDtypeStruct((E, 16), jnp.float32),
        ],
    )(ea, dd, ss, *consts, *w)


def _edge2_call(ea, rel, dd, ss, consts, w):
    return pl.pallas_call(
        _edge2_body,
        grid=(E // BE,),
        in_specs=[
            _ROWB(BE, EFD), _ROWB(BE, 16), _ROWB(BE, 2 * HID), _ROWB(BE, HID),
            _FULL(1, NRG), _FULL(EFD, RFD), _FULL(NRG, RFD),
            _FULL(HID, HEADS), _FULL(HEADS, 48), _FULL(16, 48),
            _FULL(EFD, 2 * HID), _FULL(RFD, 2 * HID),
            _FULL(HID, 2 * HID), _FULL(HID, 2 * HID), _FULL(1, 2 * HID),
            _FULL(1, HID), _FULL(1, HID), _FULL(HID, HID), _FULL(1, HID),
            _FULL(1, HID), _FULL(1, HID), _FULL(HID, HEADS), _FULL(1, HEADS),
            _FULL(1, RFD), _FULL(1, 1),
        ],
        out_specs=[_ROWB(BE, HEADS), _ROWB(BE, 48)],
        out_shape=[
            jax.ShapeDtypeStruct((E, HEADS), jnp.float32),
            jax.ShapeDtypeStruct((E, 48), jnp.float32),
        ],
    )(ea, rel, dd, ss, *consts, *w)


def _split_w1(p):
    w1 = p['W1']
    return (w1[0:EFD], w1[EFD:EFD + RFD],
            w1[EFD + RFD:EFD + RFD + HID], w1[EFD + RFD + HID:])


def _edge_weights(pk, pv, eww, ewb):
    ke, kr, kd, ks = _split_w1(pk)
    ve, vr, vd, vs = _split_w1(pv)
    w1e = jnp.concatenate([ke, ve], 1)
    w1r = jnp.concatenate([kr, vr], 1)
    w1d = jnp.concatenate([kd, vd], 1)
    w1s = jnp.concatenate([ks, vs], 1)
    b1 = jnp.concatenate([pk['b1'], pv['b1']])[None, :]
    return [w1e, w1r, w1d, w1s, b1,
            pk['g'][None, :], pk['be'][None, :], pk['W2'], pk['b2'][None, :],
            pv['g'][None, :], pv['be'][None, :], pv['W2'], pv['b2'][None, :],
            eww.reshape(1, RFD), ewb.reshape(1, 1)]


def kernel(h, x, edge_attr, edge_index, mask_ligand, params):
    p = params
    src = edge_index[0]
    dst = edge_index[1]
    xpad = jnp.concatenate([x, jnp.zeros((N, 13), jnp.float32)], 1)

    off = jnp.asarray(_OFF)[None, :]
    a4 = jnp.asarray(_A4)
    g20 = jnp.asarray(_G20)
    hs_m = jnp.asarray(_HS)
    exh = jnp.asarray(_EXH)
    p48 = jnp.asarray(_P48)
    q48 = jnp.asarray(_Q48)
    m48 = jnp.asarray(_M48)

    srcm = src.reshape(_NCH, _CHW)
    dstm = dst.reshape(_NCH, _CHW)

    q1 = _mlp128(h, p['x2h_hq'])
    dd1 = _sc_gather(jnp.concatenate([h, q1, xpad], 1), dstm, 2 * HID + 16)
    ss1 = _sc_gather(jnp.concatenate([h, xpad], 1), srcm, HID + 16)
    w1 = _edge_weights(p['x2h_hk'], p['x2h_hv'], p['x2h_ew_W'], p['x2h_ew_b'])
    ex1, wv1, rel = _edge1_call(edge_attr, dd1, ss1,
                                (off, a4, g20, hs_m, exh), w1)
    exs1, wvs1 = _sc_scatter(ex1, wv1, dstm, HID)

    po = p['x2h_out']
    hout = pl.pallas_call(
        _node1_body,
        grid=(N // BN,),
        in_specs=[
            _ROWB(BN, HEADS), _ROWB(BN, HEADS), _ROWB(BN, HID), _ROWB(BN, HID),
            _ROWB(BN, HID), _FULL(HEADS, HID),
            _FULL(HID, HID), _FULL(HID, HID), _FULL(1, HID),
            _FULL(1, HID), _FULL(1, HID), _FULL(HID, HID), _FULL(1, HID),
        ],
        out_specs=_ROWB(BN, HID),
        out_shape=jax.ShapeDtypeStruct((N, HID), jnp.float32),
    )(exs1[0], exs1[1], wvs1[0], wvs1[1], h, exh,
      po['W1'][:HID], po['W1'][HID:], po['b1'][None, :],
      po['g'][None, :], po['be'][None, :], po['W2'], po['b2'][None, :])

    q2 = _mlp128(hout, p['h2x_xq'])
    dd2 = _sc_gather(jnp.concatenate([hout, q2], 1), dstm, 2 * HID)
    ss2 = _sc_gather(hout, srcm, HID)
    w2 = _edge_weights(p['h2x_xk'], p['h2x_xv'], p['h2x_ew_W'], p['h2x_ew_b'])
    ex2, wv2 = _edge2_call(edge_attr, rel, dd2, ss2,
                           (off, a4, g20, hs_m, p48, q48), w2)
    exs2, wvs2 = _sc_scatter(ex2, wv2, dstm, 48)

    x8 = pl.pallas_call(
        _node2_body,
        grid=(N // BN,),
        in_specs=[
            _ROWB(BN, HEADS), _ROWB(BN, HEADS), _ROWB(BN, 48), _ROWB(BN, 48),
            _ROWB(BN, 16), _ROWB(BN, 8),
            _FULL(HEADS, 48), _FULL(48, 8),
        ],
        out_specs=_ROWB(BN, 8),
        out_shape=jax.ShapeDtypeStruct((N, 8), jnp.float32),
    )(exs2[0], exs2[1], wvs2[0], wvs2[1], xpad,
      jnp.broadcast_to(mask_ligand[:, None], (N, 8)), p48, m48)

    return (hout, x8[:, :3])


# speedup vs baseline: 1.1668x; 1.0400x over previous
import functools
import numpy as np
import jax
import jax.numpy as jnp
from jax import lax
from jax.experimental import pallas as pl
from jax.experimental.pallas import tpu as pltpu
from jax.experimental.pallas import tpu_sc as plsc

N = 10000
E = 160000
HID = 128
HEADS = 16
DH = 8
NRG = 20
EFD = 4
RFD = NRG * 4

BE = 2000
BN = 2000

_OFF = np.linspace(0.0, 10.0, NRG).astype(np.float32)
_COEFF = np.float32(-0.5 / (_OFF[1] - _OFF[0]) ** 2)
_A4 = np.zeros((EFD, RFD), np.float32)
_G20 = np.zeros((NRG, RFD), np.float32)
for _a in range(EFD):
    for _g in range(NRG):
        _A4[_a, _a * NRG + _g] = 1.0
        _G20[_g, _a * NRG + _g] = 1.0
_HS = np.zeros((HID, HEADS), np.float32)
_EXH = np.zeros((HEADS, HID), np.float32)
for _h in range(HEADS):
    for _j in range(DH):
        _HS[_h * DH + _j, _h] = 1.0
        _EXH[_h, _h * DH + _j] = 1.0
_P48 = np.zeros((HEADS, 48), np.float32)
_Q48 = np.zeros((16, 48), np.float32)
_M48 = np.zeros((48, 8), np.float32)
for _h in range(HEADS):
    for _c in range(3):
        _P48[_h, 3 * _h + _c] = 1.0
        _Q48[_c, 3 * _h + _c] = 1.0
        _M48[3 * _h + _c, _c] = 1.0 / HEADS
_ISQ = np.float32(1.0 / np.sqrt(DH))


_CHW = 128
_CH2 = 256
_NCH = E // _CH2
_NW = 32

_SC_MESH = dict(core_axis_name="c", subcore_axis_name="s")
_SC_PARAMS = pltpu.CompilerParams(use_tc_tiling_on_sc=False)


def _sc_gather(table, idxm, d):
    @functools.partial(
        pl.kernel,
        out_type=jax.ShapeDtypeStruct((E, d), jnp.float32),
        mesh=plsc.VectorSubcoreMesh(**_SC_MESH),
        compiler_params=_SC_PARAMS,
        scratch_types=[
            pltpu.VMEM((2, _CHW), jnp.int32),
            pltpu.VMEM((_CH2, d), jnp.float32),
            pltpu.SemaphoreType.DMA,
            pltpu.SemaphoreType.DMA,
        ],
    )
    def k(table_hbm, idxm_hbm, out_hbm, idx_v, rows_v, sem0, sem1):
        wid = lax.axis_index("s") * 2 + lax.axis_index("c")
        nch = (_NCH - wid + (_NW - 1)) // _NW

        def body(t, carry):
            ch = wid + t * _NW
            pltpu.sync_copy(idxm_hbm.at[ch], idx_v)
            pltpu.async_copy(table_hbm.at[idx_v.at[0]],
                             rows_v.at[pl.ds(0, _CHW)], sem0)
            pltpu.async_copy(table_hbm.at[idx_v.at[1]],
                             rows_v.at[pl.ds(_CHW, _CHW)], sem1)
            pltpu.make_async_copy(table_hbm.at[idx_v.at[0]],
                                  rows_v.at[pl.ds(0, _CHW)], sem0).wait()
            pltpu.make_async_copy(table_hbm.at[idx_v.at[1]],
                                  rows_v.at[pl.ds(_CHW, _CHW)], sem1).wait()
            pltpu.sync_copy(rows_v, out_hbm.at[pl.ds(ch * _CH2, _CH2)])
            return carry

        lax.fori_loop(0, nch, body, 0)

    return k(table, idxm)


def _sc_scatter(exw, wv, idxm, d):
    @functools.partial(
        pl.kernel,
        out_type=(jax.ShapeDtypeStruct((2, N, 16), jnp.float32),
                  jax.ShapeDtypeStruct((2, N, d), jnp.float32)),
        mesh=plsc.VectorSubcoreMesh(**_SC_MESH),
        compiler_params=_SC_PARAMS,
        scratch_types=[
            pltpu.VMEM((2, _CHW), jnp.int32),
            pltpu.VMEM((_CH2, 16), jnp.float32),
            pltpu.VMEM((_CH2, d), jnp.float32),
            pltpu.VMEM_SHARED((N, 16), jnp.float32),
            pltpu.VMEM_SHARED((N, d), jnp.float32),
        ],
    )
    def k(ex_hbm, wv_hbm, idxm_hbm, zex_hbm, zwv_hbm, oex_hbm, owv_hbm,
          idx_v, exb, wvb, tex, twv):
        cid = lax.axis_index("c")
        sid = lax.axis_index("s")
        wid = sid * 2 + cid

        @pl.when(sid == 0)
        def _init():
            pltpu.sync_copy(zex_hbm, tex)
            pltpu.sync_copy(zwv_hbm, twv)

        plsc.subcore_barrier()
        nch = (_NCH - wid + (_NW - 1)) // _NW

        def body(t, carry):
            ch = wid + t * _NW
            pltpu.sync_copy(idxm_hbm.at[ch], idx_v)
            pltpu.sync_copy(ex_hbm.at[pl.ds(ch * _CH2, _CH2)], exb)
            pltpu.sync_copy(wv_hbm.at[pl.ds(ch * _CH2, _CH2)], wvb)
            pltpu.sync_copy(exb.at[pl.ds(0, _CHW)],
                            tex.at[idx_v.at[0]], add=True)
            pltpu.sync_copy(wvb.at[pl.ds(0, _CHW)],
                            twv.at[idx_v.at[0]], add=True)
            pltpu.sync_copy(exb.at[pl.ds(_CHW, _CHW)],
                            tex.at[idx_v.at[1]], add=True)
            pltpu.sync_copy(wvb.at[pl.ds(_CHW, _CHW)],
                            twv.at[idx_v.at[1]], add=True)
            return carry

        lax.fori_loop(0, nch, body, 0)
        plsc.subcore_barrier()

        @pl.when(sid == 0)
        def _dump():
            pltpu.sync_copy(tex, oex_hbm.at[cid])
            pltpu.sync_copy(twv, owv_hbm.at[cid])

    zex = jnp.zeros((N, 16), jnp.float32)
    zwv = jnp.zeros((N, d), jnp.float32)
    return k(exw, wv, idxm, zex, zwv)


def _ln(v, g, b):
    mu = jnp.mean(v, -1, keepdims=True)
    var = jnp.mean((v - mu) ** 2, -1, keepdims=True)
    return (v - mu) / jnp.sqrt(var + 1e-5) * g + b


def _dot(a, b):
    return jnp.dot(a, b, preferred_element_type=jnp.float32)


def _mlp128_body(h_ref, w1, b1, g1, be1, w2, b2, o_ref):
    v = _dot(h_ref[...], w1[...]) + b1[...]
    v = jnp.maximum(_ln(v, g1[...], be1[...]), 0.0)
    o_ref[...] = _dot(v, w2[...]) + b2[...]


def _mlp128(hin, p):
    return pl.pallas_call(
        _mlp128_body,
        grid=(N // BN,),
        in_specs=[
            pl.BlockSpec((BN, HID), lambda i: (i, 0)),
            pl.BlockSpec((HID, HID), lambda i: (0, 0)),
            pl.BlockSpec((1, HID), lambda i: (0, 0)),
            pl.BlockSpec((1, HID), lambda i: (0, 0)),
            pl.BlockSpec((1, HID), lambda i: (0, 0)),
            pl.BlockSpec((HID, HID), lambda i: (0, 0)),
            pl.BlockSpec((1, HID), lambda i: (0, 0)),
        ],
        out_specs=pl.BlockSpec((BN, HID), lambda i: (i, 0)),
        out_shape=jax.ShapeDtypeStruct((N, HID), jnp.float32),
    )(hin, p['W1'], p['b1'][None, :], p['g'][None, :], p['be'][None, :],
      p['W2'], p['b2'][None, :])


def _rfeat(ea, rel, off, a4, g20):
    d2 = jnp.sum(rel * rel, -1, keepdims=True)
    dist = jnp.sqrt(d2)
    sm = jnp.exp(_COEFF * (dist - off) ** 2)
    return _dot(ea, a4) * _dot(sm, g20)


def _edge1_body(ea_ref, dd_ref, ss_ref,
                off, a4, g20, hs_m, exh,
                w1e, w1r, w1d, w1s, b1,
                gk, bek, w2k, b2k, gv, bev, w2v, b2v,
                eww, ewb,
                ex_ref, wv_ref, rel_ref):
    ea = ea_ref[...]
    hd = dd_ref[:, 0:HID]
    qd = dd_ref[:, HID:2 * HID]
    xd = dd_ref[:, 2 * HID:2 * HID + 16]
    hs = ss_ref[:, 0:HID]
    xs = ss_ref[:, HID:HID + 16]
    rel = xd - xs
    rf = _rfeat(ea, rel, off[...], a4[...], g20[...])
    pre = (_dot(ea, w1e[...]) + _dot(rf, w1r[...])
           + _dot(hd, w1d[...]) + _dot(hs, w1s[...])) + b1[...]
    pk = pre[:, :HID]
    pv = pre[:, HID:]
    k = _dot(jnp.maximum(_ln(pk, gk[...], bek[...]), 0.0), w2k[...]) + b2k[...]
    v = _dot(jnp.maximum(_ln(pv, gv[...], bev[...]), 0.0), w2v[...]) + b2v[...]
    ewl = jnp.sum(rf * eww[...], -1, keepdims=True) + ewb[...]
    v = v * (1.0 / (1.0 + jnp.exp(-ewl)))
    ex = jnp.exp(_dot(qd * k, hs_m[...]) * _ISQ)
    ex_ref[...] = ex
    wv_ref[...] = _dot(ex, exh[...]) * v
    rel_ref[...] = rel


def _edge2_body(ea_ref, rel_ref, dd_ref, ss_ref,
                off, a4, g20, hs_m, p48, q48,
                w1e, w1r, w1d, w1s, b1,
                gk, bek, w2k, b2k, gv, bev, w2v, b2v,
                eww, ewb,
                ex_ref, wv_ref):
    ea = ea_ref[...]
    rel = rel_ref[...]
    hd = dd_ref[:, 0:HID]
    qd = dd_ref[:, HID:2 * HID]
    hs = ss_ref[...]
    rf = _rfeat(ea, rel, off[...], a4[...], g20[...])
    pre = (_dot(ea, w1e[...]) + _dot(rf, w1r[...])
           + _dot(hd, w1d[...]) + _dot(hs, w1s[...])) + b1[...]
    pk = pre[:, :HID]
    pv = pre[:, HID:]
    k = _dot(jnp.maximum(_ln(pk, gk[...], bek[...]), 0.0), w2k[...]) + b2k[...]
    v = _dot(jnp.maximum(_ln(pv, gv[...], bev[...]), 0.0), w2v[...]) + b2v[...]
    ewl = jnp.sum(rf * eww[...], -1, keepdims=True) + ewb[...]
    v = v * (1.0 / (1.0 + jnp.exp(-ewl)))
    ex = jnp.exp(_dot(qd * k, hs_m[...]) * _ISQ)
    ex_ref[...] = ex
    wv_ref[...] = _dot(ex * v, p48[...]) * _dot(rel, q48[...])


def _node1_body(ex0_ref, ex1_ref, wv0_ref, wv1_ref, h_ref, exh,
                w1a, w1b, b1, g1, be1, w2, b2, ho_ref):
    exb = _dot(ex0_ref[...] + ex1_ref[...], exh[...])
    out1 = (wv0_ref[...] + wv1_ref[...]) / (exb + 1e-16)
    pre = _dot(out1, w1a[...]) + _dot(h_ref[...], w1b[...]) + b1[...]
    o = _dot(jnp.maximum(_ln(pre, g1[...], be1[...]), 0.0), w2[...]) + b2[...]
    ho_ref[...] = o + h_ref[...]


def _node2_body(ex0_ref, ex1_ref, wv0_ref, wv1_ref, x_ref, m_ref,
                p48, m48, xo_ref):
    den = _dot(ex0_ref[...] + ex1_ref[...], p48[...])
    o = (wv0_ref[...] + wv1_ref[...]) / (den + 1e-16)
    delta = _dot(o, m48[...])
    xo_ref[...] = x_ref[:, :8] + delta * m_ref[...]


_FULL = lambda r, c: pl.BlockSpec((r, c), lambda i: (0, 0))
_ROWB = lambda b, c: pl.BlockSpec((b, c), lambda i: (i, 0))


def _edge1_call(ea, dd, ss, consts, w):
    return pl.pallas_call(
        _edge1_body,
        grid=(E // BE,),
        in_specs=[
            _ROWB(BE, EFD), _ROWB(BE, 2 * HID + 16), _ROWB(BE, HID + 16),
            _FULL(1, NRG), _FULL(EFD, RFD), _FULL(NRG, RFD),
            _FULL(HID, HEADS), _FULL(HEADS, HID),
            _FULL(EFD, 2 * HID), _FULL(RFD, 2 * HID),
            _FULL(HID, 2 * HID), _FULL(HID, 2 * HID), _FULL(1, 2 * HID),
            _FULL(1, HID), _FULL(1, HID), _FULL(HID, HID), _FULL(1, HID),
            _FULL(1, HID), _FULL(1, HID), _FULL(HID, HID), _FULL(1, HID),
            _FULL(1, RFD), _FULL(1, 1),
        ],
        out_specs=[_ROWB(BE, HEADS), _ROWB(BE, HID), _ROWB(BE, 16)],
        out_shape=[
            jax.ShapeDtypeStruct((E, HEADS), jnp.float32),
            jax.ShapeDtypeStruct((E, HID), jnp.float32),
            jax.ShapeDtypeStruct((E, 16), jnp.float32),
        ],
    )(ea, dd, ss, *consts, *w)


def _edge2_call(ea, rel, dd, ss, consts, w):
    return pl.pallas_call(
        _edge2_body,
        grid=(E // BE,),
        in_specs=[
            _ROWB(BE, EFD), _ROWB(BE, 16), _ROWB(BE, 2 * HID), _ROWB(BE, HID),
            _FULL(1, NRG), _FULL(EFD, RFD), _FULL(NRG, RFD),
            _FULL(HID, HEADS), _FULL(HEADS, 48), _FULL(16, 48),
            _FULL(EFD, 2 * HID), _FULL(RFD, 2 * HID),
            _FULL(HID, 2 * HID), _FULL(HID, 2 * HID), _FULL(1, 2 * HID),
            _FULL(1, HID), _FULL(1, HID), _FULL(HID, HID), _FULL(1, HID),
            _FULL(1, HID), _FULL(1, HID), _FULL(HID, HEADS), _FULL(1, HEADS),
            _FULL(1, RFD), _FULL(1, 1),
        ],
        out_specs=[_ROWB(BE, HEADS), _ROWB(BE, 48)],
        out_shape=[
            jax.ShapeDtypeStruct((E, HEADS), jnp.float32),
            jax.ShapeDtypeStruct((E, 48), jnp.float32),
        ],
    )(ea, rel, dd, ss, *consts, *w)


def _split_w1(p):
    w1 = p['W1']
    return (w1[0:EFD], w1[EFD:EFD + RFD],
            w1[EFD + RFD:EFD + RFD + HID], w1[EFD + RFD + HID:])


def _edge_weights(pk, pv, eww, ewb):
    ke, kr, kd, ks = _split_w1(pk)
    ve, vr, vd, vs = _split_w1(pv)
    w1e = jnp.concatenate([ke, ve], 1)
    w1r = jnp.concatenate([kr, vr], 1)
    w1d = jnp.concatenate([kd, vd], 1)
    w1s = jnp.concatenate([ks, vs], 1)
    b1 = jnp.concatenate([pk['b1'], pv['b1']])[None, :]
    return [w1e, w1r, w1d, w1s, b1,
            pk['g'][None, :], pk['be'][None, :], pk['W2'], pk['b2'][None, :],
            pv['g'][None, :], pv['be'][None, :], pv['W2'], pv['b2'][None, :],
            eww.reshape(1, RFD), ewb.reshape(1, 1)]


def kernel(h, x, edge_attr, edge_index, mask_ligand, params):
    p = params
    src = edge_index[0]
    dst = edge_index[1]
    xpad = jnp.concatenate([x, jnp.zeros((N, 13), jnp.float32)], 1)

    off = jnp.asarray(_OFF)[None, :]
    a4 = jnp.asarray(_A4)
    g20 = jnp.asarray(_G20)
    hs_m = jnp.asarray(_HS)
    exh = jnp.asarray(_EXH)
    p48 = jnp.asarray(_P48)
    q48 = jnp.asarray(_Q48)
    m48 = jnp.asarray(_M48)

    srcm = src.reshape(_NCH, 2, _CHW)
    dstm = dst.reshape(_NCH, 2, _CHW)

    q1 = _mlp128(h, p['x2h_hq'])
    dd1 = _sc_gather(jnp.concatenate([h, q1, xpad], 1), dstm, 2 * HID + 16)
    ss1 = _sc_gather(jnp.concatenate([h, xpad], 1), srcm, HID + 16)
    w1 = _edge_weights(p['x2h_hk'], p['x2h_hv'], p['x2h_ew_W'], p['x2h_ew_b'])
    ex1, wv1, rel = _edge1_call(edge_attr, dd1, ss1,
                                (off, a4, g20, hs_m, exh), w1)
    exs1, wvs1 = _sc_scatter(ex1, wv1, dstm, HID)

    po = p['x2h_out']
    hout = pl.pallas_call(
        _node1_body,
        grid=(N // BN,),
        in_specs=[
            _ROWB(BN, HEADS), _ROWB(BN, HEADS), _ROWB(BN, HID), _ROWB(BN, HID),
            _ROWB(BN, HID), _FULL(HEADS, HID),
            _FULL(HID, HID), _FULL(HID, HID), _FULL(1, HID),
            _FULL(1, HID), _FULL(1, HID), _FULL(HID, HID), _FULL(1, HID),
        ],
        out_specs=_ROWB(BN, HID),
        out_shape=jax.ShapeDtypeStruct((N, HID), jnp.float32),
    )(exs1[0], exs1[1], wvs1[0], wvs1[1], h, exh,
      po['W1'][:HID], po['W1'][HID:], po['b1'][None, :],
      po['g'][None, :], po['be'][None, :], po['W2'], po['b2'][None, :])

    q2 = _mlp128(hout, p['h2x_xq'])
    dd2 = _sc_gather(jnp.concatenate([hout, q2], 1), dstm, 2 * HID)
    ss2 = _sc_gather(hout, srcm, HID)
    w2 = _edge_weights(p['h2x_xk'], p['h2x_xv'], p['h2x_ew_W'], p['h2x_ew_b'])
    ex2, wv2 = _edge2_call(edge_attr, rel, dd2, ss2,
                           (off, a4, g20, hs_m, p48, q48), w2)
    exs2, wvs2 = _sc_scatter(ex2, wv2, dstm, 48)

    x8 = pl.pallas_call(
        _node2_body,
        grid=(N // BN,),
        in_specs=[
            _ROWB(BN, HEADS), _ROWB(BN, HEADS), _ROWB(BN, 48), _ROWB(BN, 48),
            _ROWB(BN, 16), _ROWB(BN, 8),
            _FULL(HEADS, 48), _FULL(48, 8),
        ],
        out_specs=_ROWB(BN, 8),
        out_shape=jax.ShapeDtypeStruct((N, 8), jnp.float32),
    )(exs2[0], exs2[1], wvs2[0], wvs2[1], xpad,
      jnp.broadcast_to(mask_ligand[:, None], (N, 8)), p48, m48)

    return (hout, x8[:, :3])
